# baseline (speedup 1.0000x reference)
import functools

import jax
import jax.numpy as jnp
from jax import lax
from jax.experimental import pallas as pl
from jax.experimental.pallas import tpu as pltpu
from jax.experimental.pallas import tpu_sc as plsc

_VOCAB = 1000000
_HIDDEN = 64
_B = 4096
_L = 200
_N = _B * _L
_NW = 32
_PER_W = _N // _NW
_G = 128
_NG = _PER_W // _G
_K = 4
_NH = _NG // _K


def _body(idx_hbm, table_hbm, out_hbm, idx_v, rows_v, gsem, wsem):
    wid = lax.axis_index("s") * 2 + lax.axis_index("c")
    pltpu.sync_copy(idx_hbm.at[pl.ds(wid * _NG, _NG)], idx_v)
    grp_base = wid * _NG

    def outer(j2, carry):
        for jj in range(2):
            j = j2 * 2 + jj
            b0 = jj * _K
            half = rows_v.at[pl.ds(b0, _K)]

            @pl.when(j2 > 0)
            def _drain_prev_write():
                pltpu.make_async_copy(
                    half,
                    out_hbm.at[pl.ds(grp_base, _K), :, pl.ds(0, _HIDDEN)],
                    wsem,
                ).wait()

            for b in range(_K):
                g = j * _K + b
                pltpu.async_copy(
                    table_hbm.at[idx_v.at[g]], rows_v.at[b0 + b], gsem
                )
            for b in range(_K):
                pltpu.make_async_copy(
                    table_hbm.at[idx_v.at[0]], rows_v.at[b0], gsem
                ).wait()
            pltpu.async_copy(
                half,
                out_hbm.at[pl.ds(grp_base + j * _K, _K), :, pl.ds(0, _HIDDEN)],
                wsem,
            )
        return carry

    lax.fori_loop(0, _NH // 2, outer, 0)
    for jj in range(2):
        pltpu.make_async_copy(
            rows_v.at[pl.ds(jj * _K, _K)],
            out_hbm.at[pl.ds(grp_base, _K), :, pl.ds(0, _HIDDEN)],
            wsem,
        ).wait()


@jax.jit
def _run(idx, table):
    mesh = plsc.VectorSubcoreMesh(core_axis_name="c", subcore_axis_name="s")
    f = functools.partial(
        pl.kernel,
        mesh=mesh,
        out_type=jax.ShapeDtypeStruct((_N // _G, _G, 2 * _HIDDEN), jnp.float32),
        scratch_types=[
            pltpu.VMEM((_NG, _G), jnp.int32),
            pltpu.VMEM((2 * _K, _G, _HIDDEN), jnp.float32),
            pltpu.SemaphoreType.DMA,
            pltpu.SemaphoreType.DMA,
        ],
        compiler_params=pltpu.CompilerParams(use_tc_tiling_on_sc=False),
    )(_body)
    return f(idx, table)


def kernel(input_ids, attention_mask, embedding_weight):
    del attention_mask
    idx = input_ids.reshape(-1).astype(jnp.int32).reshape(_N // _G, _G)
    out = _run(idx, embedding_weight)
    return out[:, :, :_HIDDEN].reshape(_B, _L, _HIDDEN)
